# Initial kernel scaffold; baseline (speedup 1.0000x reference)
import jax
import jax.numpy as jnp
from jax.experimental import pallas as pl


def kernel(x, edge_index, unused, batch, Wl1, Wr1, b1, Wl2, Wr2, b2, Wl3, Wr3, b3, Wl4, Wr4, b4, Wl5, Wr5, b5, fc1_W, fc1_b, fc2_W, fc2_b):
    raise NotImplementedError("write your pallas kernel here")


# speedup vs baseline: 2.6473x; 2.6473x over previous
import functools

import jax
import jax.numpy as jnp
from jax import lax
from jax.experimental import pallas as pl
from jax.experimental.pallas import tpu as pltpu
from jax.experimental.pallas import tpu_sc as plsc

N = 10000
E = 320000
F = 128
H = 128
G = 128

NC = 2
NS = 16
NW = NC * NS
CHUNK = 128
CPT = 80
IBLK = 16
NIB = CPT // IBLK
EPAD = NW * CPT * CHUNK
NACC = 10112
ROWS_Z = NACC // NS
ROWS_W = 632
ROWS_W_LAST = N - 15 * ROWS_W

@functools.lru_cache(maxsize=None)
def _mesh():
    return plsc.VectorSubcoreMesh(core_axis_name="c", subcore_axis_name="s")


def _sc_agg_body(x_hbm, srci_hbm, dsti_hbm, zeros_hbm, out_hbm,
                 srcv, dstv, rows, acc):
    c = lax.axis_index("c")
    s = lax.axis_index("s")
    wid = c * NS + s

    pltpu.sync_copy(zeros_hbm.at[pl.ds(s * ROWS_Z, ROWS_Z)],
                    acc.at[pl.ds(s * ROWS_Z, ROWS_Z)])
    plsc.subcore_barrier()

    @pl.loop(0, NIB)
    def _(bk):
        pltpu.sync_copy(srci_hbm.at[wid, pl.ds(bk * IBLK, IBLK)], srcv)
        pltpu.sync_copy(dsti_hbm.at[wid, pl.ds(bk * IBLK, IBLK)], dstv)

        @pl.loop(0, IBLK)
        def _(j):
            pltpu.sync_copy(x_hbm.at[srcv.at[j]], rows)
            pltpu.sync_copy(rows, acc.at[dstv.at[j]], add=True)

    plsc.subcore_barrier()

    @pl.when(s < NS - 1)
    def _():
        pltpu.sync_copy(acc.at[pl.ds(s * ROWS_W, ROWS_W)],
                        out_hbm.at[c].at[pl.ds(s * ROWS_W, ROWS_W)])

    @pl.when(s == NS - 1)
    def _():
        pltpu.sync_copy(acc.at[pl.ds(15 * ROWS_W, ROWS_W_LAST)],
                        out_hbm.at[c].at[pl.ds(15 * ROWS_W, ROWS_W_LAST)])


@functools.lru_cache(maxsize=None)
def _sc_agg_kernel():
    return pl.kernel(
        _sc_agg_body,
        out_type=[jax.ShapeDtypeStruct((NC, N, F), jnp.float32)],
        mesh=_mesh(),
        scratch_types=[
            pltpu.VMEM((IBLK, CHUNK), jnp.int32),
            pltpu.VMEM((IBLK, CHUNK), jnp.int32),
            pltpu.VMEM((CHUNK, F), jnp.float32),
            pltpu.VMEM_SHARED((NACC, F), jnp.float32),
        ],
    )


BLK = 2000
NBLK = N // BLK


def _tc_layer_body(agg_ref, deg_ref, h_ref, wl_ref, wr_ref, b_ref, o_ref):
    p = agg_ref[0] + agg_ref[1]
    d = deg_ref[0, :, 0:1] + deg_ref[1, :, 0:1]
    a = p / jnp.maximum(d, 1.0)
    acc = jnp.dot(a, wl_ref[...], preferred_element_type=jnp.float32)
    acc = acc + jnp.dot(h_ref[...], wr_ref[...], preferred_element_type=jnp.float32)
    acc = acc + b_ref[...]
    o_ref[...] = jnp.maximum(acc, 0.0)


def _tc_layer(agg2, deg2, h, Wl, Wr, b2d):
    return pl.pallas_call(
        _tc_layer_body,
        grid=(NBLK,),
        in_specs=[
            pl.BlockSpec((NC, BLK, F), lambda i: (0, i, 0)),
            pl.BlockSpec((NC, BLK, F), lambda i: (0, i, 0)),
            pl.BlockSpec((BLK, F), lambda i: (i, 0)),
            pl.BlockSpec((F, H), lambda i: (0, 0)),
            pl.BlockSpec((F, H), lambda i: (0, 0)),
            pl.BlockSpec((1, H), lambda i: (0, 0)),
        ],
        out_specs=pl.BlockSpec((BLK, H), lambda i: (i, 0)),
        out_shape=jax.ShapeDtypeStruct((N, H), jnp.float32),
    )(agg2, deg2, h, Wl, Wr, b2d)


def _tc_final_body(agg_ref, deg_ref, h_ref, wl_ref, wr_ref, b_ref, bat_ref,
                   fc1_ref, fc1b_ref, fc2_ref, fc2b_ref, o_ref,
                   acc_ref, cnt_ref):
    i = pl.program_id(0)

    @pl.when(i == 0)
    def _():
        acc_ref[...] = jnp.zeros_like(acc_ref)
        cnt_ref[...] = jnp.zeros_like(cnt_ref)

    p = agg_ref[0] + agg_ref[1]
    d = deg_ref[0, :, 0:1] + deg_ref[1, :, 0:1]
    a = p / jnp.maximum(d, 1.0)
    h5 = jnp.dot(a, wl_ref[...], preferred_element_type=jnp.float32)
    h5 = h5 + jnp.dot(h_ref[...], wr_ref[...], preferred_element_type=jnp.float32)
    h5 = jnp.maximum(h5 + b_ref[...], 0.0)

    gids = lax.broadcasted_iota(jnp.int32, (BLK, G), 1)
    onehot = (gids == bat_ref[...]).astype(jnp.float32)
    acc_ref[...] += lax.dot_general(
        onehot, h5, (((0,), (0,)), ((), ())), preferred_element_type=jnp.float32)
    cnt_ref[...] += lax.dot_general(
        onehot, jnp.ones((BLK, 128), jnp.float32), (((0,), (0,)), ((), ())),
        preferred_element_type=jnp.float32)

    @pl.when(i == NBLK - 1)
    def _():
        pooled = acc_ref[...] / jnp.maximum(cnt_ref[...], 1.0)
        h2 = jnp.dot(pooled, fc1_ref[...], preferred_element_type=jnp.float32)
        h2 = jnp.maximum(h2 + fc1b_ref[...], 0.0)
        logits = jnp.dot(h2, fc2_ref[...], preferred_element_type=jnp.float32)
        o_ref[...] = jax.nn.sigmoid(logits + fc2b_ref[...])


def _tc_final(agg2, deg2, h, Wl, Wr, b2d, batchb, fc1_W, fc1b2d, fc2p, fc2b2d):
    return pl.pallas_call(
        _tc_final_body,
        grid=(NBLK,),
        in_specs=[
            pl.BlockSpec((NC, BLK, F), lambda i: (0, i, 0)),
            pl.BlockSpec((NC, BLK, F), lambda i: (0, i, 0)),
            pl.BlockSpec((BLK, F), lambda i: (i, 0)),
            pl.BlockSpec((F, H), lambda i: (0, 0)),
            pl.BlockSpec((F, H), lambda i: (0, 0)),
            pl.BlockSpec((1, H), lambda i: (0, 0)),
            pl.BlockSpec((BLK, G), lambda i: (i, 0)),
            pl.BlockSpec((H, H), lambda i: (0, 0)),
            pl.BlockSpec((1, H), lambda i: (0, 0)),
            pl.BlockSpec((H, 128), lambda i: (0, 0)),
            pl.BlockSpec((1, 128), lambda i: (0, 0)),
        ],
        out_specs=pl.BlockSpec((G, 128), lambda i: (0, 0)),
        out_shape=jax.ShapeDtypeStruct((G, 128), jnp.float32),
        scratch_shapes=[
            pltpu.VMEM((G, H), jnp.float32),
            pltpu.VMEM((G, 128), jnp.float32),
        ],
    )(agg2, deg2, h, Wl, Wr, b2d, batchb, fc1_W, fc1b2d, fc2p, fc2b2d)


def kernel(x, edge_index, unused, batch, Wl1, Wr1, b1, Wl2, Wr2, b2, Wl3,
           Wr3, b3, Wl4, Wr4, b4, Wl5, Wr5, b5, fc1_W, fc1_b, fc2_W, fc2_b):
    src = edge_index[:, 0]
    dst = edge_index[:, 1]
    pad = EPAD - E
    src_p = jnp.concatenate([src, jnp.zeros((pad,), jnp.int32)]).reshape(
        NW, CPT, CHUNK)
    dst_p = jnp.concatenate([dst, jnp.full((pad,), N, jnp.int32)]).reshape(
        NW, CPT, CHUNK)
    zeros_hbm = jnp.zeros((NACC, F), jnp.float32)
    ones_nf = jnp.ones((N, F), jnp.float32)
    batchb = jnp.broadcast_to(batch[:, None], (N, G))
    fc2p = jnp.concatenate([fc2_W, jnp.zeros((H, 127), jnp.float32)], axis=1)
    fc2b2d = jnp.broadcast_to(fc2_b.reshape(1, 1), (1, 128))
    b2ds = [b.reshape(1, H) for b in (b1, b2, b3, b4, b5)]
    fc1b2d = fc1_b.reshape(1, H)

    (deg2,) = _sc_agg_kernel()(ones_nf, src_p, dst_p, zeros_hbm)
    (agg2,) = _sc_agg_kernel()(x, src_p, dst_p, zeros_hbm)
    h = _tc_layer(agg2, deg2, x, Wl1, Wr1, b2ds[0])
    for Wl, Wr, b2d in ((Wl2, Wr2, b2ds[1]), (Wl3, Wr3, b2ds[2]),
                        (Wl4, Wr4, b2ds[3])):
        (agg2,) = _sc_agg_kernel()(h, src_p, dst_p, zeros_hbm)
        h = _tc_layer(agg2, deg2, h, Wl, Wr, b2d)
    (agg2,) = _sc_agg_kernel()(h, src_p, dst_p, zeros_hbm)
    res = _tc_final(agg2, deg2, h, Wl5, Wr5, b2ds[4], batchb, fc1_W, fc1b2d,
                    fc2p, fc2b2d)
    return res[:, 0]


# speedup vs baseline: 3.2293x; 1.2198x over previous
import dataclasses
import functools

import jax
import jax.numpy as jnp
from jax import lax
from jax.experimental import pallas as pl
from jax.experimental.pallas import tpu as pltpu
from jax.experimental.pallas import tpu_sc as plsc

N = 10000
E = 320000
F = 128
H = 128
G = 128

NC = 2
NS = 16
NW = NC * NS
CHUNK = 128
CPT = 80
IBLK = 16
NIB = CPT // IBLK
EPAD = NW * CPT * CHUNK
NHIST = 10016
NACC = 10112
ROWS_Z = NACC // NS
ROWS_W = 632
ROWS_W_LAST = N - 15 * ROWS_W

@functools.lru_cache(maxsize=None)
def _mesh():
    return plsc.VectorSubcoreMesh(core_axis_name="c", subcore_axis_name="s")


def _sc_agg_body(with_hist, *refs):
    if with_hist:
        (x_hbm, srci_hbm, dsti_hbm, zeros_hbm, out_hbm, hist_hbm,
         srcv, dstv, rows_a, rows_b, acc, hist, sem_a, sem_b) = refs
    else:
        (x_hbm, srci_hbm, dsti_hbm, zeros_hbm, out_hbm,
         srcv, dstv, rows_a, rows_b, acc, sem_a, sem_b) = refs
    c = lax.axis_index("c")
    s = lax.axis_index("s")
    wid = c * NS + s

    pltpu.sync_copy(zeros_hbm.at[pl.ds(s * ROWS_Z, ROWS_Z)],
                    acc.at[pl.ds(s * ROWS_Z, ROWS_Z)])
    if with_hist:
        @pl.loop(0, NHIST // 16)
        def _(i):
            hist[pl.ds(i * 16, 16)] = jnp.zeros((16,), jnp.float32)
    plsc.subcore_barrier()

    def g_start(j, buf, sem):
        pltpu.make_async_copy(x_hbm.at[srcv.at[j]], buf, sem).start()

    def g_wait(j, buf, sem):
        pltpu.make_async_copy(x_hbm.at[srcv.at[j]], buf, sem).wait()

    @pl.loop(0, NIB)
    def _(bk):
        pltpu.sync_copy(srci_hbm.at[wid, pl.ds(bk * IBLK, IBLK)], srcv)
        pltpu.sync_copy(dsti_hbm.at[wid, pl.ds(bk * IBLK, IBLK)], dstv)
        g_start(0, rows_a, sem_a)

        @pl.loop(0, IBLK // 2)
        def _(p):
            j0 = p * 2
            j1 = j0 + 1
            g_start(j1, rows_b, sem_b)
            g_wait(j0, rows_a, sem_a)
            pltpu.sync_copy(rows_a, acc.at[dstv.at[j0]], add=True)

            @pl.when(p < IBLK // 2 - 1)
            def _():
                g_start(j0 + 2, rows_a, sem_a)

            g_wait(j1, rows_b, sem_b)
            pltpu.sync_copy(rows_b, acc.at[dstv.at[j1]], add=True)

            if with_hist:
                @pl.loop(0, CHUNK // 16)
                def _(kk):
                    ones = jnp.ones((16,), jnp.float32)
                    plsc.addupdate_scatter(
                        hist, [dstv[j0, pl.ds(kk * 16, 16)]], ones)
                    plsc.addupdate_scatter(
                        hist, [dstv[j1, pl.ds(kk * 16, 16)]], ones)

    plsc.subcore_barrier()

    @pl.when(s < NS - 1)
    def _():
        pltpu.sync_copy(acc.at[pl.ds(s * ROWS_W, ROWS_W)],
                        out_hbm.at[c].at[pl.ds(s * ROWS_W, ROWS_W)])

    @pl.when(s == NS - 1)
    def _():
        pltpu.sync_copy(acc.at[pl.ds(15 * ROWS_W, ROWS_W_LAST)],
                        out_hbm.at[c].at[pl.ds(15 * ROWS_W, ROWS_W_LAST)])

    if with_hist:
        pltpu.sync_copy(hist, hist_hbm.at[wid])


def _sc_compiler_params():
    cp = pltpu.CompilerParams()
    if "needs_layout_passes" in pltpu.CompilerParams.__dataclass_fields__:
        cp = dataclasses.replace(cp, needs_layout_passes=False)
    return cp


@functools.lru_cache(maxsize=None)
def _sc_agg_kernel(with_hist):
    out_type = [jax.ShapeDtypeStruct((NC, N, F), jnp.float32)]
    scratch = [
        pltpu.VMEM((IBLK, CHUNK), jnp.int32),
        pltpu.VMEM((IBLK, CHUNK), jnp.int32),
        pltpu.VMEM((CHUNK, F), jnp.float32),
        pltpu.VMEM((CHUNK, F), jnp.float32),
        pltpu.VMEM_SHARED((NACC, F), jnp.float32),
    ]
    if with_hist:
        out_type.append(jax.ShapeDtypeStruct((NW, NHIST), jnp.float32))
        scratch.append(pltpu.VMEM((NHIST,), jnp.float32))
    scratch += [pltpu.SemaphoreType.DMA, pltpu.SemaphoreType.DMA]
    return pl.kernel(
        functools.partial(_sc_agg_body, with_hist),
        out_type=out_type,
        mesh=_mesh(),
        compiler_params=_sc_compiler_params() if with_hist else None,
        scratch_types=scratch,
    )


BLK = 2000
NBLK = N // BLK


def _tc_layer_body(agg_ref, deg_ref, h_ref, wl_ref, wr_ref, b_ref, o_ref):
    p = agg_ref[0] + agg_ref[1]
    d = jnp.dot(deg_ref[...], jnp.ones((NW, 1), jnp.float32),
                preferred_element_type=jnp.float32)
    a = p / jnp.maximum(d, 1.0)
    acc = jnp.dot(a, wl_ref[...], preferred_element_type=jnp.float32)
    acc = acc + jnp.dot(h_ref[...], wr_ref[...], preferred_element_type=jnp.float32)
    acc = acc + b_ref[...]
    o_ref[...] = jnp.maximum(acc, 0.0)


def _tc_layer(agg2, deg2, h, Wl, Wr, b2d):
    return pl.pallas_call(
        _tc_layer_body,
        grid=(NBLK,),
        in_specs=[
            pl.BlockSpec((NC, BLK, F), lambda i: (0, i, 0)),
            pl.BlockSpec((BLK, NW), lambda i: (i, 0)),
            pl.BlockSpec((BLK, F), lambda i: (i, 0)),
            pl.BlockSpec((F, H), lambda i: (0, 0)),
            pl.BlockSpec((F, H), lambda i: (0, 0)),
            pl.BlockSpec((1, H), lambda i: (0, 0)),
        ],
        out_specs=pl.BlockSpec((BLK, H), lambda i: (i, 0)),
        out_shape=jax.ShapeDtypeStruct((N, H), jnp.float32),
    )(agg2, deg2, h, Wl, Wr, b2d)


def _tc_final_body(agg_ref, deg_ref, h_ref, wl_ref, wr_ref, b_ref, bat_ref,
                   fc1_ref, fc1b_ref, fc2_ref, fc2b_ref, o_ref,
                   acc_ref, cnt_ref):
    i = pl.program_id(0)

    @pl.when(i == 0)
    def _():
        acc_ref[...] = jnp.zeros_like(acc_ref)
        cnt_ref[...] = jnp.zeros_like(cnt_ref)

    p = agg_ref[0] + agg_ref[1]
    d = jnp.dot(deg_ref[...], jnp.ones((NW, 1), jnp.float32),
                preferred_element_type=jnp.float32)
    a = p / jnp.maximum(d, 1.0)
    h5 = jnp.dot(a, wl_ref[...], preferred_element_type=jnp.float32)
    h5 = h5 + jnp.dot(h_ref[...], wr_ref[...], preferred_element_type=jnp.float32)
    h5 = jnp.maximum(h5 + b_ref[...], 0.0)

    gids = lax.broadcasted_iota(jnp.int32, (BLK, G), 1)
    onehot = (gids == bat_ref[...]).astype(jnp.float32)
    acc_ref[...] += lax.dot_general(
        onehot, h5, (((0,), (0,)), ((), ())), preferred_element_type=jnp.float32)
    cnt_ref[...] += lax.dot_general(
        onehot, jnp.ones((BLK, 128), jnp.float32), (((0,), (0,)), ((), ())),
        preferred_element_type=jnp.float32)

    @pl.when(i == NBLK - 1)
    def _():
        pooled = acc_ref[...] / jnp.maximum(cnt_ref[...], 1.0)
        h2 = jnp.dot(pooled, fc1_ref[...], preferred_element_type=jnp.float32)
        h2 = jnp.maximum(h2 + fc1b_ref[...], 0.0)
        logits = jnp.dot(h2, fc2_ref[...], preferred_element_type=jnp.float32)
        o_ref[...] = jax.nn.sigmoid(logits + fc2b_ref[...])


def _tc_final(agg2, deg2, h, Wl, Wr, b2d, batchb, fc1_W, fc1b2d, fc2p, fc2b2d):
    return pl.pallas_call(
        _tc_final_body,
        grid=(NBLK,),
        in_specs=[
            pl.BlockSpec((NC, BLK, F), lambda i: (0, i, 0)),
            pl.BlockSpec((BLK, NW), lambda i: (i, 0)),
            pl.BlockSpec((BLK, F), lambda i: (i, 0)),
            pl.BlockSpec((F, H), lambda i: (0, 0)),
            pl.BlockSpec((F, H), lambda i: (0, 0)),
            pl.BlockSpec((1, H), lambda i: (0, 0)),
            pl.BlockSpec((BLK, G), lambda i: (i, 0)),
            pl.BlockSpec((H, H), lambda i: (0, 0)),
            pl.BlockSpec((1, H), lambda i: (0, 0)),
            pl.BlockSpec((H, 128), lambda i: (0, 0)),
            pl.BlockSpec((1, 128), lambda i: (0, 0)),
        ],
        out_specs=pl.BlockSpec((G, 128), lambda i: (0, 0)),
        out_shape=jax.ShapeDtypeStruct((G, 128), jnp.float32),
        scratch_shapes=[
            pltpu.VMEM((G, H), jnp.float32),
            pltpu.VMEM((G, 128), jnp.float32),
        ],
    )(agg2, deg2, h, Wl, Wr, b2d, batchb, fc1_W, fc1b2d, fc2p, fc2b2d)


def kernel(x, edge_index, unused, batch, Wl1, Wr1, b1, Wl2, Wr2, b2, Wl3,
           Wr3, b3, Wl4, Wr4, b4, Wl5, Wr5, b5, fc1_W, fc1_b, fc2_W, fc2_b):
    src = edge_index[:, 0]
    dst = edge_index[:, 1]
    pad = EPAD - E
    src_p = jnp.concatenate([src, jnp.zeros((pad,), jnp.int32)]).reshape(
        NW, CPT, CHUNK)
    dst_p = jnp.concatenate([dst, jnp.full((pad,), N, jnp.int32)]).reshape(
        NW, CPT, CHUNK)
    zeros_hbm = jnp.zeros((NACC, F), jnp.float32)
    batchb = jnp.broadcast_to(batch[:, None], (N, G))
    fc2p = jnp.concatenate([fc2_W, jnp.zeros((H, 127), jnp.float32)], axis=1)
    fc2b2d = jnp.broadcast_to(fc2_b.reshape(1, 1), (1, 128))
    b2ds = [b.reshape(1, H) for b in (b1, b2, b3, b4, b5)]
    fc1b2d = fc1_b.reshape(1, H)

    agg2, histp = _sc_agg_kernel(True)(x, src_p, dst_p, zeros_hbm)
    degp = histp[:, :N].T
    h = _tc_layer(agg2, degp, x, Wl1, Wr1, b2ds[0])
    for Wl, Wr, b2d in ((Wl2, Wr2, b2ds[1]), (Wl3, Wr3, b2ds[2]),
                        (Wl4, Wr4, b2ds[3])):
        (agg2,) = _sc_agg_kernel(False)(h, src_p, dst_p, zeros_hbm)
        h = _tc_layer(agg2, degp, h, Wl, Wr, b2d)
    (agg2,) = _sc_agg_kernel(False)(h, src_p, dst_p, zeros_hbm)
    res = _tc_final(agg2, degp, h, Wl5, Wr5, b2ds[4], batchb, fc1_W, fc1b2d,
                    fc2p, fc2b2d)
    return res[:, 0]
